# Initial kernel scaffold; baseline (speedup 1.0000x reference)
import jax
import jax.numpy as jnp
from jax.experimental import pallas as pl


def kernel(x, edge_index, W0, W1, W2, conv_bias, lin_W, lin_b):
    raise NotImplementedError("write your pallas kernel here")


# speedup vs baseline: 22.9921x; 22.9921x over previous
import functools

import jax
import jax.numpy as jnp
from jax import lax
from jax.experimental import pallas as pl
from jax.experimental.pallas import tpu as pltpu
from jax.experimental.pallas import tpu_sc as plsc

_N = 10000
_E = 320000
_D = 128
_H = 16
_OUT = 128

_NC = 2
_NS = 16
_NW = _NC * _NS
_L = 16

_B = 128
_EPT = _E // _NW
_NCHUNK = -(-_EPT // _B)
_EPTP = _NCHUNK * _B
_NP = 10240
_RPT = _NP // _NS
_SRT = 160


def _make_prop(C, gather):
  mesh = plsc.VectorSubcoreMesh(
      core_axis_name="c", subcore_axis_name="s",
      num_cores=_NC, num_subcores=_NS)
  scratch = [
      pltpu.VMEM((_EPTP,), jnp.int32) if gather else
      pltpu.VMEM((_L,), jnp.int32),
      pltpu.VMEM((_NCHUNK, _B), jnp.int32),
      pltpu.VMEM((2 if gather else 1, _B, C), jnp.float32),
      pltpu.VMEM((_SRT, C), jnp.float32),
      pltpu.VMEM_SHARED((_NP, C), jnp.float32),
      pltpu.SemaphoreType.DMA,
      pltpu.SemaphoreType.DMA,
  ]
  _TAIL = _EPT - (_NCHUNK - 1) * _B

  def body(row_hbm, col_hbm, src_hbm, out_hbm,
           rowidx, colidx, msg, stage, acc, gsem, psem):
    cid = lax.axis_index("c")
    sid = lax.axis_index("s")
    wid = sid * _NC + cid
    base_e = wid * _EPT

    zf = jnp.zeros((_L,), jnp.float32)
    def zbody(r, carry):
      for kk in range(C // _L):
        stage[r, pl.ds(kk * _L, _L)] = zf
      return carry
    lax.fori_loop(0, _SRT, zbody, 0)
    def zcopy(b, carry):
      pltpu.sync_copy(stage, acc.at[pl.ds(sid * _RPT + b * _SRT, _SRT)])
      return carry
    lax.fori_loop(0, _RPT // _SRT, zcopy, 0)

    def cstart(j, carry):
      pltpu.make_async_copy(
          col_hbm.at[pl.ds(base_e + j * _B, _B)], colidx.at[j], psem).start()
      return carry
    lax.fori_loop(0, _NCHUNK - 1, cstart, 0)
    cp_tail = pltpu.make_async_copy(
        col_hbm.at[pl.ds(base_e + (_NCHUNK - 1) * _B, _TAIL)],
        colidx.at[_NCHUNK - 1].at[pl.ds(0, _TAIL)], psem)
    cp_tail.start()
    if gather:
      cp_r = pltpu.make_async_copy(
          row_hbm.at[pl.ds(base_e, _EPT)], rowidx.at[pl.ds(0, _EPT)], psem)
      cp_r.start()
      for kk in range((_EPTP - _EPT) // _L):
        rowidx[pl.ds(_EPT + kk * _L, _L)] = jnp.zeros((_L,), jnp.int32)
    else:
      of = jnp.ones((_L,), jnp.float32)
      def obody(r, carry):
        for kk in range(C // _L):
          msg[0, r, pl.ds(kk * _L, _L)] = of
        return carry
      lax.fori_loop(0, _B, obody, 0)
    for kk in range((_B - _TAIL) // _L):
      colidx[_NCHUNK - 1, pl.ds(_TAIL + kk * _L, _L)] = jnp.full(
          (_L,), _N, jnp.int32)
    def cwait(j, carry):
      pltpu.make_async_copy(
          col_hbm.at[pl.ds(base_e + j * _B, _B)], colidx.at[j], psem).wait()
      return carry
    lax.fori_loop(0, _NCHUNK - 1, cwait, 0)
    cp_tail.wait()
    if gather:
      cp_r.wait()

    plsc.subcore_barrier()

    if gather:
      def gdesc(j, p):
        return pltpu.make_async_copy(
            src_hbm.at[rowidx.at[pl.ds(j * _B, _B)]], msg.at[p], gsem)

      gdesc(0, 0).start()

      def ebody(j, carry):
        p = lax.rem(j, 2)
        gdesc(j, p).wait()
        @pl.when(j + 1 < _NCHUNK)
        def _start_next():
          gdesc(j + 1, 1 - p).start()
        pltpu.sync_copy(msg.at[p], acc.at[colidx.at[j]], add=True)
        return carry
      lax.fori_loop(0, _NCHUNK, ebody, 0)
    else:
      def ebody(j, carry):
        pltpu.sync_copy(msg.at[0], acc.at[colidx.at[j]], add=True)
        return carry
      lax.fori_loop(0, _NCHUNK, ebody, 0)

    plsc.subcore_barrier()

    def wcopy(b, carry):
      base = sid * _RPT + b * _SRT
      pltpu.sync_copy(acc.at[pl.ds(base, _SRT)], stage)
      pltpu.sync_copy(stage, out_hbm.at[cid].at[pl.ds(base, _SRT)])
      return carry
    lax.fori_loop(0, _RPT // _SRT, wcopy, 0)

  return functools.partial(
      pl.kernel,
      out_type=jax.ShapeDtypeStruct((_NC, _NP, C), jnp.float32),
      mesh=mesh, scratch_types=scratch,
      compiler_params=pltpu.CompilerParams(use_tc_tiling_on_sc=False))(body)


_BLK = 128
_G = -(-_N // _BLK)


def _dense1(x, wcat, h0, h1):
  def body(x_ref, w_ref, h0_ref, h1_ref, dis_ref, out0_ref, yp_ref):
    deg = h0_ref[:, 0:1] + h1_ref[:, 0:1] + 1.0
    dis = lax.rsqrt(deg)
    dis_ref[...] = dis
    xw = jnp.dot(x_ref[...], w_ref[...],
                 preferred_element_type=jnp.float32,
                 precision=lax.Precision.HIGHEST)
    out0_ref[...] = xw[:, 0:_H]
    yp_ref[...] = xw[:, _H:3 * _H] * dis
  return pl.pallas_call(
      body,
      grid=(_G,),
      in_specs=[pl.BlockSpec((_BLK, _D), lambda i: (i, 0)),
                pl.BlockSpec((_D, 3 * _H), lambda i: (0, 0)),
                pl.BlockSpec((_BLK, _L), lambda i: (i, 0)),
                pl.BlockSpec((_BLK, _L), lambda i: (i, 0))],
      out_specs=[pl.BlockSpec((_BLK, 1), lambda i: (i, 0)),
                 pl.BlockSpec((_BLK, _H), lambda i: (i, 0)),
                 pl.BlockSpec((_BLK, 2 * _H), lambda i: (i, 0))],
      out_shape=[jax.ShapeDtypeStruct((_N, 1), jnp.float32),
                 jax.ShapeDtypeStruct((_N, _H), jnp.float32),
                 jax.ShapeDtypeStruct((_N, 2 * _H), jnp.float32)],
  )(x, wcat, h0, h1)


def _dense2(a0, a1, yp, dis):
  def body(a0_ref, a1_ref, yp_ref, dis_ref, out1_ref, zp_ref):
    dis = dis_ref[...]
    p1 = (a0_ref[...] + a1_ref[...] + yp_ref[...]) * dis
    out1_ref[...] = p1[:, 0:_H]
    zp_ref[...] = p1[:, _H:2 * _H] * dis
  return pl.pallas_call(
      body,
      grid=(_G,),
      in_specs=[pl.BlockSpec((_BLK, 2 * _H), lambda i: (i, 0)),
                pl.BlockSpec((_BLK, 2 * _H), lambda i: (i, 0)),
                pl.BlockSpec((_BLK, 2 * _H), lambda i: (i, 0)),
                pl.BlockSpec((_BLK, 1), lambda i: (i, 0))],
      out_specs=[pl.BlockSpec((_BLK, _H), lambda i: (i, 0)),
                 pl.BlockSpec((_BLK, _H), lambda i: (i, 0))],
      out_shape=[jax.ShapeDtypeStruct((_N, _H), jnp.float32),
                 jax.ShapeDtypeStruct((_N, _H), jnp.float32)],
  )(a0, a1, yp, dis)


def _dense3(a0, a1, zp, dis, out0, out1, cb, lw, lb):
  def body(a0_ref, a1_ref, zp_ref, dis_ref, o0_ref, o1_ref, cb_ref, lw_ref,
           lb_ref, out_ref):
    out2 = (a0_ref[...] + a1_ref[...] + zp_ref[...]) * dis_ref[...]
    h = jnp.concatenate([o0_ref[...], o1_ref[...], out2], axis=1)
    h = jnp.maximum(h + cb_ref[...], 0.0)
    o = jnp.dot(h, lw_ref[...],
                preferred_element_type=jnp.float32,
                precision=lax.Precision.HIGHEST)
    out_ref[...] = o + lb_ref[...]
  return pl.pallas_call(
      body,
      grid=(_G,),
      in_specs=[pl.BlockSpec((_BLK, _H), lambda i: (i, 0)),
                pl.BlockSpec((_BLK, _H), lambda i: (i, 0)),
                pl.BlockSpec((_BLK, _H), lambda i: (i, 0)),
                pl.BlockSpec((_BLK, 1), lambda i: (i, 0)),
                pl.BlockSpec((_BLK, _H), lambda i: (i, 0)),
                pl.BlockSpec((_BLK, _H), lambda i: (i, 0)),
                pl.BlockSpec((1, 3 * _H), lambda i: (0, 0)),
                pl.BlockSpec((3 * _H, _OUT), lambda i: (0, 0)),
                pl.BlockSpec((1, _OUT), lambda i: (0, 0))],
      out_specs=pl.BlockSpec((_BLK, _OUT), lambda i: (i, 0)),
      out_shape=jax.ShapeDtypeStruct((_N, _OUT), jnp.float32),
  )(a0, a1, zp, dis, out0, out1, cb, lw, lb)


_hist = _make_prop(_L, gather=False)
_prop32 = _make_prop(2 * _H, gather=True)
_prop16 = _make_prop(_H, gather=True)


def kernel(x, edge_index, W0, W1, W2, conv_bias, lin_W, lin_b):
  row = edge_index[0]
  col = edge_index[1]
  wcat = jnp.concatenate([W0, W1, W2], axis=1)

  hist = _hist(row, col, x)
  h0 = hist[0, :_N, :]
  h1 = hist[1, :_N, :]
  dis, out0, yp = _dense1(x, wcat, h0, h1)

  a1 = _prop32(row, col, yp)
  out1, zp = _dense2(a1[0, :_N], a1[1, :_N], yp, dis)

  a2 = _prop16(row, col, zp)
  out = _dense3(a2[0, :_N], a2[1, :_N], zp, dis, out0, out1,
                conv_bias.reshape(1, 3 * _H), lin_W, lin_b.reshape(1, _OUT))
  return out
